# baseline (speedup 1.0000x reference)
import functools

import jax
import jax.numpy as jnp
import numpy as np
from jax import lax
from jax.experimental import pallas as pl
from jax.experimental.pallas import tpu as pltpu
from jax.experimental.pallas import tpu_sc as plsc

B, C, S = 32, 32, 8192
BMIN, BMAX = 30, 50
START_IDX, END_IDX = 4500, 5250
MASK_RATIO = 5 * 0.5 / 9.0

LANES = 16
IDX_CHUNK = 128


def _gen_blocks(rng, available_indices, total_mask_length):
    min_size, max_size = BMIN, BMAX
    mask_positions = []
    remaining = total_mask_length
    arr = np.array(available_indices)
    rng.shuffle(arr)
    available_indices = arr.tolist()
    while remaining >= min_size and available_indices:
        block_size = min(
            max_size,
            remaining,
            int(rng.integers(min_size, min(max_size, remaining) + 1)),
        )
        a = np.asarray(available_indices)
        n = len(a) - block_size + 1
        if n <= 0:
            valid_starts = []
        else:
            ok = np.ones(n, dtype=bool)
            base = a[:n]
            for j in range(1, block_size):
                ok &= a[j : j + n] == base + j
            valid_starts = np.nonzero(ok)[0].tolist()
        if not valid_starts:
            positions = available_indices[:remaining]
            mask_positions.extend(positions[:block_size])
            remaining -= len(positions[:block_size])
            break
        start_idx = valid_starts[int(rng.integers(len(valid_starts)))]
        block_positions = available_indices[start_idx : start_idx + block_size]
        mask_positions.extend(block_positions)
        remaining -= block_size
        for pos in block_positions:
            available_indices.remove(pos)
    return sorted(set(mask_positions))


def _build_index_tables():
    rng = np.random.default_rng(0)
    available = list(range(0, START_IDX)) + list(range(END_IDX, S))
    total_mask_length = int(len(available) * MASK_RATIO)
    iota = np.arange(S)
    p_rows, g_rows, k_max = [], [], 0
    for _ in range(B):
        src = np.arange(S)
        if total_mask_length >= BMIN and rng.random() < 1.0:
            for pos in _gen_blocks(rng, list(available), total_mask_length):
                if pos > 0:
                    src[pos] = src[pos - 1]
                elif pos < S - 1:
                    src[pos] = src[pos + 1]
        p = np.nonzero(src != iota)[0]
        g = src[p]
        assert np.all(src[g] == g)
        p_rows.append(p)
        g_rows.append(g)
        k_max = max(k_max, len(p))
    k_pad = max(LANES, -(-k_max // LANES) * LANES)
    pad = START_IDX
    p_tab = np.full((B, k_pad), pad, np.int32)
    g_tab = np.full((B, k_pad), pad, np.int32)
    for b in range(B):
        p_tab[b, : len(p_rows[b])] = p_rows[b]
        g_tab[b, : len(g_rows[b])] = g_rows[b]
    offs = (np.arange(ROWS_PER_CHUNK, dtype=np.int32) * S)[None, :, None]
    p_loc = (p_tab[:, None, :] + offs).reshape(B, -1)
    g_loc = (g_tab[:, None, :] + offs).reshape(B, -1)
    return p_loc.astype(np.int32), g_loc.astype(np.int32), p_loc.shape[1]


ROWS_PER_CHUNK = 4
CHUNK_WORDS = ROWS_PER_CHUNK * S
NUM_CHUNKS = C // ROWS_PER_CHUNK
NBUF = 3

_P_TAB, _G_TAB, _K_LOC = _build_index_tables()


def _sc_body(x_hbm, p_hbm, g_hbm, out_hbm, pv, gv, bufs, in_sems, out_sems):
    b = lax.axis_index("s") * 2 + lax.axis_index("c")
    pltpu.sync_copy(p_hbm.at[b], pv)
    pltpu.sync_copy(g_hbm.at[b], gv)

    def chunk_in(i):
        return x_hbm.at[b, pl.ds(i * CHUNK_WORDS, CHUNK_WORDS)]

    def chunk_out(i):
        return out_hbm.at[b, pl.ds(i * CHUNK_WORDS, CHUNK_WORDS)]

    def fixup(buf):
        def fix(j, _):
            g = gv[pl.ds(j * LANES, LANES)]
            p = pv[pl.ds(j * LANES, LANES)]
            vals = plsc.load_gather(buf, [g])
            plsc.store_scatter(buf, [p], vals)
            return 0

        lax.fori_loop(0, _K_LOC // LANES, fix, 0)

    pltpu.async_copy(chunk_in(0), bufs[0], in_sems[0])
    pltpu.async_copy(chunk_in(1), bufs[1], in_sems[1])
    for i in range(NUM_CHUNKS):
        bi = i % NBUF
        pltpu.make_async_copy(chunk_in(i), bufs[bi], in_sems[bi]).wait()
        fixup(bufs[bi])
        pltpu.async_copy(bufs[bi], chunk_out(i), out_sems[bi])
        nxt = i + 2
        if nxt < NUM_CHUNKS:
            nbi = nxt % NBUF
            if i >= 1:
                pltpu.make_async_copy(bufs[nbi], chunk_out(i - 1), out_sems[nbi]).wait()
            pltpu.async_copy(chunk_in(nxt), bufs[nbi], in_sems[nbi])
    for i in range(NUM_CHUNKS - 2, NUM_CHUNKS):
        bi = i % NBUF
        pltpu.make_async_copy(bufs[bi], chunk_out(i), out_sems[bi]).wait()


def kernel(x):
    p_tab = jnp.asarray(_P_TAB)
    g_tab = jnp.asarray(_G_TAB)
    mesh = plsc.VectorSubcoreMesh(core_axis_name="c", subcore_axis_name="s")
    run = functools.partial(
        pl.kernel,
        mesh=mesh,
        out_type=jax.ShapeDtypeStruct((B, C * S), jnp.float32),
        scratch_types=[
            pltpu.VMEM((_K_LOC,), jnp.int32),
            pltpu.VMEM((_K_LOC,), jnp.int32),
            [pltpu.VMEM((CHUNK_WORDS,), jnp.float32) for _ in range(NBUF)],
            [pltpu.SemaphoreType.DMA for _ in range(NBUF)],
            [pltpu.SemaphoreType.DMA for _ in range(NBUF)],
        ],
        compiler_params=pltpu.CompilerParams(needs_layout_passes=False),
    )(_sc_body)
    return run(x.reshape(B, C * S), p_tab, g_tab).reshape(B, C, S)


# speedup vs baseline: 2.2142x; 2.2142x over previous
import functools

import jax
import jax.numpy as jnp
import numpy as np
from jax import lax
from jax.experimental import pallas as pl
from jax.experimental.pallas import tpu as pltpu
from jax.experimental.pallas import tpu_sc as plsc

B, C, S = 32, 32, 8192
BMIN, BMAX = 30, 50
START_IDX, END_IDX = 4500, 5250
MASK_RATIO = 5 * 0.5 / 9.0

LANES = 16
IDX_CHUNK = 128


def _gen_blocks(rng, available_indices, total_mask_length):
    min_size, max_size = BMIN, BMAX
    mask_positions = []
    remaining = total_mask_length
    arr = np.array(available_indices)
    rng.shuffle(arr)
    available_indices = arr.tolist()
    while remaining >= min_size and available_indices:
        block_size = min(
            max_size,
            remaining,
            int(rng.integers(min_size, min(max_size, remaining) + 1)),
        )
        a = np.asarray(available_indices)
        n = len(a) - block_size + 1
        if n <= 0:
            valid_starts = []
        else:
            ok = np.ones(n, dtype=bool)
            base = a[:n]
            for j in range(1, block_size):
                ok &= a[j : j + n] == base + j
            valid_starts = np.nonzero(ok)[0].tolist()
        if not valid_starts:
            positions = available_indices[:remaining]
            mask_positions.extend(positions[:block_size])
            remaining -= len(positions[:block_size])
            break
        start_idx = valid_starts[int(rng.integers(len(valid_starts)))]
        block_positions = available_indices[start_idx : start_idx + block_size]
        mask_positions.extend(block_positions)
        remaining -= block_size
        for pos in block_positions:
            available_indices.remove(pos)
    return sorted(set(mask_positions))


def _build_index_tables():
    rng = np.random.default_rng(0)
    available = list(range(0, START_IDX)) + list(range(END_IDX, S))
    total_mask_length = int(len(available) * MASK_RATIO)
    iota = np.arange(S)
    p_rows, g_rows, k_max = [], [], 0
    for _ in range(B):
        src = np.arange(S)
        if total_mask_length >= BMIN and rng.random() < 1.0:
            for pos in _gen_blocks(rng, list(available), total_mask_length):
                if pos > 0:
                    src[pos] = src[pos - 1]
                elif pos < S - 1:
                    src[pos] = src[pos + 1]
        p = np.nonzero(src != iota)[0]
        g = src[p]
        assert np.all(src[g] == g)
        p_rows.append(p)
        g_rows.append(g)
        k_max = max(k_max, len(p))
    k_pad = max(LANES, -(-k_max // LANES) * LANES)
    pad = START_IDX
    p_tab = np.full((B, k_pad), pad, np.int32)
    g_tab = np.full((B, k_pad), pad, np.int32)
    for b in range(B):
        p_tab[b, : len(p_rows[b])] = p_rows[b]
        g_tab[b, : len(g_rows[b])] = g_rows[b]
    return p_tab, g_tab, k_pad


NBUF = 6
KAHEAD = 3

_P_TAB, _G_TAB, _K_PAD = _build_index_tables()


def _sc_body(x_hbm, p_hbm, g_hbm, out_hbm, pv, gv, bufs, in_sems, out_sems):
    b = lax.axis_index("s") * 2 + lax.axis_index("c")
    pltpu.sync_copy(p_hbm.at[b], pv)
    pltpu.sync_copy(g_hbm.at[b], gv)

    def fixup(buf):
        def fix(j, _):
            g = gv[pl.ds(j * LANES, LANES)]
            p = pv[pl.ds(j * LANES, LANES)]
            vals = plsc.load_gather(buf, [g])
            plsc.store_scatter(buf, [p], vals)
            return 0

        lax.fori_loop(0, _K_PAD // LANES, fix, 0)

    for i in range(KAHEAD):
        pltpu.async_copy(x_hbm.at[b, i], bufs[i % NBUF], in_sems[i % NBUF])
    for i in range(C):
        bi = i % NBUF
        pltpu.make_async_copy(x_hbm.at[b, i], bufs[bi], in_sems[bi]).wait()
        fixup(bufs[bi])
        pltpu.async_copy(bufs[bi], out_hbm.at[b, i], out_sems[bi])
        nxt = i + KAHEAD
        if nxt < C:
            nbi = nxt % NBUF
            prev = nxt - NBUF
            if prev >= 0:
                pltpu.make_async_copy(bufs[nbi], out_hbm.at[b, prev], out_sems[nbi]).wait()
            pltpu.async_copy(x_hbm.at[b, nxt], bufs[nbi], in_sems[nbi])
    for i in range(C - NBUF, C):
        bi = i % NBUF
        pltpu.make_async_copy(bufs[bi], out_hbm.at[b, i], out_sems[bi]).wait()


def kernel(x):
    p_tab = jnp.asarray(_P_TAB)
    g_tab = jnp.asarray(_G_TAB)
    mesh = plsc.VectorSubcoreMesh(core_axis_name="c", subcore_axis_name="s")
    run = functools.partial(
        pl.kernel,
        mesh=mesh,
        out_type=jax.ShapeDtypeStruct((B, C, S), jnp.float32),
        scratch_types=[
            pltpu.VMEM((_K_PAD,), jnp.int32),
            pltpu.VMEM((_K_PAD,), jnp.int32),
            [pltpu.VMEM((S,), jnp.float32) for _ in range(NBUF)],
            [pltpu.SemaphoreType.DMA for _ in range(NBUF)],
            [pltpu.SemaphoreType.DMA for _ in range(NBUF)],
        ],
        compiler_params=pltpu.CompilerParams(needs_layout_passes=False),
    )(_sc_body)
    return run(x, p_tab, g_tab)
